# baseline (speedup 1.0000x reference)
import functools

import jax
import jax.numpy as jnp
import numpy as np
from jax import lax
from jax.experimental import pallas as pl
from jax.experimental.pallas import tpu as pltpu
from jax.experimental.pallas import tpu_sc as plsc

NUM_SELECT = 60
NUM_BODY_POINTS = 17
_B = 32
_N = 20000
_C = 2
_NL = _N * _C
_NCH = _NL // 16
_CAP = 4096
_PAD_SEL = 64
_KP_IN = NUM_BODY_POINTS * 2
_KP_COLS = NUM_BODY_POINTS * 3
_GW = 160
_NEG = float("-inf")
_IMAX = 2**31 - 1


def _extract_top60(read_val, write_val, read_idx, nv):
  lane = lax.iota(jnp.int32, 16)

  def round_body(r, carry):
    s0, s1, s2, s3, i0, i1, i2, i3 = carry

    def max_body(k, acc):
      return jnp.maximum(acc, read_val(k))

    mx = lax.fori_loop(0, nv, max_body, jnp.full((16,), _NEG, jnp.float32))
    m = jnp.max(mx)

    def idx_body(k, acc):
      v = read_val(k)
      ii = read_idx(k)
      return jnp.minimum(acc, jnp.where(v == m, ii, _IMAX))

    mi_v = lax.fori_loop(0, nv, idx_body, jnp.full((16,), _IMAX, jnp.int32))
    mi = -jnp.max(-mi_v)

    def clear_body(k, c):
      v = read_val(k)
      ii = read_idx(k)
      write_val(k, jnp.where(ii == mi, _NEG, v))
      return c

    lax.fori_loop(0, nv, clear_body, 0)

    lane_hit = lane == (r & 15)
    slot = r >> 4
    mv = jnp.full((16,), m, jnp.float32)
    iv = jnp.full((16,), mi, jnp.int32)
    s0 = jnp.where(jnp.logical_and(lane_hit, slot == 0), mv, s0)
    s1 = jnp.where(jnp.logical_and(lane_hit, slot == 1), mv, s1)
    s2 = jnp.where(jnp.logical_and(lane_hit, slot == 2), mv, s2)
    s3 = jnp.where(jnp.logical_and(lane_hit, slot == 3), mv, s3)
    i0 = jnp.where(jnp.logical_and(lane_hit, slot == 0), iv, i0)
    i1 = jnp.where(jnp.logical_and(lane_hit, slot == 1), iv, i1)
    i2 = jnp.where(jnp.logical_and(lane_hit, slot == 2), iv, i2)
    i3 = jnp.where(jnp.logical_and(lane_hit, slot == 3), iv, i3)
    return s0, s1, s2, s3, i0, i1, i2, i3

  zf = jnp.zeros((16,), jnp.float32)
  zi = jnp.zeros((16,), jnp.int32)
  return lax.fori_loop(0, NUM_SELECT, round_body,
                       (zf, zf, zf, zf, zi, zi, zi, zi))


def _sc_body(logits_hbm, scores_hbm, labels_hbm, nidx_hbm,
             x_v, cv, ci, sc_v, lb_v, ni_v):
  b = lax.axis_index("s") * 2 + lax.axis_index("c")

  pltpu.sync_copy(logits_hbm.at[b], x_v)

  lane = lax.iota(jnp.int32, 16)
  negv = jnp.full((16,), _NEG, jnp.float32)

  def log_idx(k):
    return lane + k * 16

  def p1(k, carry):
    r0, r1, r2, r3 = carry
    v = x_v[pl.ds(k * 16, 16)]
    b0 = jnp.maximum(r0, v)
    v1 = jnp.minimum(r0, v)
    b1 = jnp.maximum(r1, v1)
    v2 = jnp.minimum(r1, v1)
    b2 = jnp.maximum(r2, v2)
    v3 = jnp.minimum(r2, v2)
    b3 = jnp.maximum(r3, v3)
    return b0, b1, b2, b3

  _, _, _, r3 = lax.fori_loop(0, _NCH, p1, (negv, negv, negv, negv))
  t = -jnp.max(-r3)
  tv = jnp.full((16,), t, jnp.float32)

  def p2(k, pos):
    v = x_v[pl.ds(k * 16, 16)]
    m = v >= tv
    ps = jnp.minimum(pos, _CAP)
    plsc.store_compressed(cv.at[pl.ds(ps, 16)], v, mask=m)
    plsc.store_compressed(ci.at[pl.ds(ps, 16)], log_idx(k), mask=m)
    return pos + jnp.max(plsc.all_reduce_population_count(m))

  cnt = lax.fori_loop(0, _NCH, p2, 0)

  ps = jnp.minimum(cnt, _CAP)
  cv[pl.ds(ps, 16)] = negv
  ci[pl.ds(ps, 16)] = jnp.full((16,), _IMAX, jnp.int32)

  def main_path(_):
    nv = (jnp.minimum(cnt, _CAP) + 15) >> 4

    def rv(k):
      return cv[pl.ds(k * 16, 16)]

    def wv(k, x):
      cv[pl.ds(k * 16, 16)] = x

    def ri(k):
      return ci[pl.ds(k * 16, 16)]

    return _extract_top60(rv, wv, ri, nv)

  def slow_path(_):
    def rv(k):
      return x_v[pl.ds(k * 16, 16)]

    def wv(k, x):
      x_v[pl.ds(k * 16, 16)] = x

    return _extract_top60(rv, wv, log_idx, _NCH)

  s0, s1, s2, s3, i0, i1, i2, i3 = lax.cond(
      cnt <= _CAP, main_path, slow_path, 0)

  one = jnp.float32(1.0)
  for s, (svreg, ivreg) in enumerate(
      ((s0, i0), (s1, i1), (s2, i2), (s3, i3))):
    sc_v[pl.ds(s * 16, 16)] = one / (one + jnp.exp(-svreg))
    lb_v[pl.ds(s * 16, 16)] = ivreg & 1
    ni_v[pl.ds(s * 16, 16)] = ivreg >> 1

  pltpu.sync_copy(sc_v, scores_hbm.at[b])
  pltpu.sync_copy(lb_v, labels_hbm.at[b])
  pltpu.sync_copy(ni_v, nidx_hbm.at[b])


def _sc_select(logits_t):
  mesh = plsc.VectorSubcoreMesh(core_axis_name="c", subcore_axis_name="s")
  f = pl.kernel(
      _sc_body,
      out_type=(
          jax.ShapeDtypeStruct((_B, _PAD_SEL), jnp.float32),
          jax.ShapeDtypeStruct((_B, _PAD_SEL), jnp.int32),
          jax.ShapeDtypeStruct((_B, _PAD_SEL), jnp.int32),
      ),
      mesh=mesh,
      compiler_params=pltpu.CompilerParams(needs_layout_passes=False),
      scratch_types=[
          pltpu.VMEM((_NL,), jnp.float32),
          pltpu.VMEM((_CAP + 16,), jnp.float32),
          pltpu.VMEM((_CAP + 16,), jnp.int32),
          pltpu.VMEM((_PAD_SEL,), jnp.float32),
          pltpu.VMEM((_PAD_SEL,), jnp.int32),
          pltpu.VMEM((_PAD_SEL,), jnp.int32),
      ],
  )
  return f(logits_t)


def _tc_body(ts_smem, kp_hbm, nidx_ref, m_ref, out_ref, slab_v, sem):
  b = pl.program_id(0)

  cp = pltpu.make_async_copy(kp_hbm.at[:, b, :], slab_v, sem)
  cp.start()
  cp.wait()

  ks = lax.broadcasted_iota(jnp.int32, (_N, _PAD_SEL), 0)
  oh = (ks == nidx_ref[0]).astype(jnp.float32)
  cols = jax.lax.dot_general(
      slab_v[...], oh, (((1,), (0,)), ((), ())),
      preferred_element_type=jnp.float32)

  rows = cols.T
  mm = jax.lax.dot_general(
      rows, m_ref[...], (((1,), (0,)), ((), ())),
      precision=lax.Precision.HIGHEST,
      preferred_element_type=jnp.float32)
  hh = ts_smem[b, 0]
  ww = ts_smem[b, 1]
  rem = lax.broadcasted_iota(jnp.int32, (_PAD_SEL, _KP_COLS), 1) % 3
  svec = jnp.where(rem == 0, ww, jnp.where(rem == 1, hh, 0.0))
  ones = jnp.where(rem == 2, 1.0, 0.0)
  out_ref[0] = mm * svec + ones


def _tc_gather(nidx, target_sizes, kp_t, mmat):
  return pl.pallas_call(
      _tc_body,
      grid_spec=pltpu.PrefetchScalarGridSpec(
          num_scalar_prefetch=1,
          grid=(_B,),
          in_specs=[
              pl.BlockSpec(memory_space=pl.ANY),
              pl.BlockSpec((1, 1, _PAD_SEL), lambda b, tref: (b, 0, 0)),
              pl.BlockSpec((_KP_IN, _KP_COLS), lambda b, tref: (0, 0)),
          ],
          out_specs=pl.BlockSpec(
              (1, _PAD_SEL, _KP_COLS), lambda b, tref: (b, 0, 0)),
          scratch_shapes=[
              pltpu.VMEM((_KP_IN, _N), jnp.float32),
              pltpu.SemaphoreType.DMA,
          ],
      ),
      out_shape=jax.ShapeDtypeStruct((_B, _PAD_SEL, _KP_COLS), jnp.float32),
  )(target_sizes, kp_t, nidx.reshape(_B, 1, _PAD_SEL), mmat)


def _build_mmat():
  m = np.zeros((_KP_IN, _KP_COLS), np.float32)
  for j in range(_KP_IN):
    m[j, 3 * (j // 2) + (j % 2)] = 1.0
  return m


@jax.jit
def _post_process(pred_logits, pred_keypoints, target_sizes):
  logits_t = pred_logits.reshape(_B, _NL)
  kp_t = jnp.transpose(pred_keypoints, (2, 0, 1))
  scores_p, labels_p, nidx = _sc_select(logits_t)
  mmat = jnp.asarray(_build_mmat())
  kp_p = _tc_gather(nidx, target_sizes, kp_t, mmat)
  return (scores_p[:, :NUM_SELECT],
          labels_p[:, :NUM_SELECT],
          kp_p[:, :NUM_SELECT, :])


def kernel(pred_logits, pred_keypoints, target_sizes):
  return _post_process(pred_logits, pred_keypoints, target_sizes)


# speedup vs baseline: 1.2724x; 1.2724x over previous
import functools

import jax
import jax.numpy as jnp
import numpy as np
from jax import lax
from jax.experimental import pallas as pl
from jax.experimental.pallas import tpu as pltpu
from jax.experimental.pallas import tpu_sc as plsc

NUM_SELECT = 60
NUM_BODY_POINTS = 17
_B = 32
_N = 20000
_C = 2
_NL = _N * _C
_NCH = _NL // 16
_CAP = 4096
_PAD_SEL = 64
_KP_IN = NUM_BODY_POINTS * 2
_KP_COLS = NUM_BODY_POINTS * 3
_GW = 160
_NEG = float("-inf")
_IMAX = 2**31 - 1


def _extract_top60(read_val, write_val, read_idx, nv):
  lane = lax.iota(jnp.int32, 16)

  def round_body(r, carry):
    s0, s1, s2, s3, i0, i1, i2, i3 = carry

    def max_body(k, acc):
      return jnp.maximum(acc, read_val(k))

    mx = lax.fori_loop(0, nv, max_body, jnp.full((16,), _NEG, jnp.float32))
    m = jnp.max(mx)

    def idx_body(k, acc):
      v = read_val(k)
      ii = read_idx(k)
      return jnp.minimum(acc, jnp.where(v == m, ii, _IMAX))

    mi_v = lax.fori_loop(0, nv, idx_body, jnp.full((16,), _IMAX, jnp.int32))
    mi = -jnp.max(-mi_v)

    def clear_body(k, c):
      v = read_val(k)
      ii = read_idx(k)
      write_val(k, jnp.where(ii == mi, _NEG, v))
      return c

    lax.fori_loop(0, nv, clear_body, 0)

    lane_hit = lane == (r & 15)
    slot = r >> 4
    mv = jnp.full((16,), m, jnp.float32)
    iv = jnp.full((16,), mi, jnp.int32)
    s0 = jnp.where(jnp.logical_and(lane_hit, slot == 0), mv, s0)
    s1 = jnp.where(jnp.logical_and(lane_hit, slot == 1), mv, s1)
    s2 = jnp.where(jnp.logical_and(lane_hit, slot == 2), mv, s2)
    s3 = jnp.where(jnp.logical_and(lane_hit, slot == 3), mv, s3)
    i0 = jnp.where(jnp.logical_and(lane_hit, slot == 0), iv, i0)
    i1 = jnp.where(jnp.logical_and(lane_hit, slot == 1), iv, i1)
    i2 = jnp.where(jnp.logical_and(lane_hit, slot == 2), iv, i2)
    i3 = jnp.where(jnp.logical_and(lane_hit, slot == 3), iv, i3)
    return s0, s1, s2, s3, i0, i1, i2, i3

  zf = jnp.zeros((16,), jnp.float32)
  zi = jnp.zeros((16,), jnp.int32)
  return lax.fori_loop(0, NUM_SELECT, round_body,
                       (zf, zf, zf, zf, zi, zi, zi, zi))


def _sc_body(logits_hbm, scores_hbm, labels_hbm, nidx_hbm,
             x_v, cv, ci, sc_v, lb_v, ni_v):
  b = lax.axis_index("s") * 2 + lax.axis_index("c")

  pltpu.sync_copy(logits_hbm.at[b], x_v)

  lane = lax.iota(jnp.int32, 16)
  negv = jnp.full((16,), _NEG, jnp.float32)

  def log_idx(k):
    return lane + k * 16

  def p1(k, carry):
    r0, r1, r2, r3 = carry
    v = x_v[pl.ds(k * 16, 16)]
    b0 = jnp.maximum(r0, v)
    v1 = jnp.minimum(r0, v)
    b1 = jnp.maximum(r1, v1)
    v2 = jnp.minimum(r1, v1)
    b2 = jnp.maximum(r2, v2)
    v3 = jnp.minimum(r2, v2)
    b3 = jnp.maximum(r3, v3)
    return b0, b1, b2, b3

  _, _, _, r3 = lax.fori_loop(0, _NCH, p1, (negv, negv, negv, negv))
  t = -jnp.max(-r3)
  tv = jnp.full((16,), t, jnp.float32)

  def p2(k, pos):
    v = x_v[pl.ds(k * 16, 16)]
    m = v >= tv
    ps = jnp.minimum(pos, _CAP)
    plsc.store_compressed(cv.at[pl.ds(ps, 16)], v, mask=m)
    plsc.store_compressed(ci.at[pl.ds(ps, 16)], log_idx(k), mask=m)
    return pos + jnp.max(plsc.all_reduce_population_count(m))

  cnt = lax.fori_loop(0, _NCH, p2, 0)

  ps = jnp.minimum(cnt, _CAP)
  cv[pl.ds(ps, 16)] = negv
  ci[pl.ds(ps, 16)] = jnp.full((16,), _IMAX, jnp.int32)

  def main_path(_):
    nv = (jnp.minimum(cnt, _CAP) + 15) >> 4

    def rv(k):
      return cv[pl.ds(k * 16, 16)]

    def wv(k, x):
      cv[pl.ds(k * 16, 16)] = x

    def ri(k):
      return ci[pl.ds(k * 16, 16)]

    return _extract_top60(rv, wv, ri, nv)

  def slow_path(_):
    def rv(k):
      return x_v[pl.ds(k * 16, 16)]

    def wv(k, x):
      x_v[pl.ds(k * 16, 16)] = x

    return _extract_top60(rv, wv, log_idx, _NCH)

  s0, s1, s2, s3, i0, i1, i2, i3 = lax.cond(
      cnt <= _CAP, main_path, slow_path, 0)

  one = jnp.float32(1.0)
  for s, (svreg, ivreg) in enumerate(
      ((s0, i0), (s1, i1), (s2, i2), (s3, i3))):
    sc_v[pl.ds(s * 16, 16)] = one / (one + jnp.exp(-svreg))
    lb_v[pl.ds(s * 16, 16)] = ivreg & 1
    ni_v[pl.ds(s * 16, 16)] = ivreg >> 1

  pltpu.sync_copy(sc_v, scores_hbm.at[b])
  pltpu.sync_copy(lb_v, labels_hbm.at[b])
  pltpu.sync_copy(ni_v, nidx_hbm.at[b])


def _sc_select(logits_t):
  mesh = plsc.VectorSubcoreMesh(core_axis_name="c", subcore_axis_name="s")
  f = pl.kernel(
      _sc_body,
      out_type=(
          jax.ShapeDtypeStruct((_B, _PAD_SEL), jnp.float32),
          jax.ShapeDtypeStruct((_B, _PAD_SEL), jnp.int32),
          jax.ShapeDtypeStruct((_B, _PAD_SEL), jnp.int32),
      ),
      mesh=mesh,
      compiler_params=pltpu.CompilerParams(needs_layout_passes=False),
      scratch_types=[
          pltpu.VMEM((_NL,), jnp.float32),
          pltpu.VMEM((_CAP + 16,), jnp.float32),
          pltpu.VMEM((_CAP + 16,), jnp.int32),
          pltpu.VMEM((_PAD_SEL,), jnp.float32),
          pltpu.VMEM((_PAD_SEL,), jnp.int32),
          pltpu.VMEM((_PAD_SEL,), jnp.int32),
      ],
  )
  return f(logits_t)


def _tc_body(ts_smem, kp_hbm, nidx_ref, m_ref, out_ref,
             slab0, slab1, sem0, sem1):
  b = pl.program_id(0)

  @pl.when(b == 0)
  def _():
    pltpu.make_async_copy(kp_hbm.at[:, 0, :], slab0, sem0).start()

  def step(cur_slab, cur_sem, nxt_slab, nxt_sem):
    @pl.when(b + 1 < _B)
    def _():
      pltpu.make_async_copy(kp_hbm.at[:, b + 1, :], nxt_slab, nxt_sem).start()

    pltpu.make_async_copy(kp_hbm.at[:, b, :], cur_slab, cur_sem).wait()

    ks = lax.broadcasted_iota(jnp.int32, (_N, _PAD_SEL), 0)
    oh = (ks == nidx_ref[0]).astype(jnp.float32)
    cols = jax.lax.dot_general(
        cur_slab[...], oh, (((1,), (0,)), ((), ())),
        preferred_element_type=jnp.float32)

    rows = cols.T
    mm = jax.lax.dot_general(
        rows, m_ref[...], (((1,), (0,)), ((), ())),
        precision=lax.Precision.HIGHEST,
        preferred_element_type=jnp.float32)
    hh = ts_smem[b, 0]
    ww = ts_smem[b, 1]
    rem = lax.broadcasted_iota(jnp.int32, (_PAD_SEL, _KP_COLS), 1) % 3
    svec = jnp.where(rem == 0, ww, jnp.where(rem == 1, hh, 0.0))
    ones = jnp.where(rem == 2, 1.0, 0.0)
    out_ref[0] = mm * svec + ones

  @pl.when((b & 1) == 0)
  def _():
    step(slab0, sem0, slab1, sem1)

  @pl.when((b & 1) == 1)
  def _():
    step(slab1, sem1, slab0, sem0)


def _tc_gather(nidx, target_sizes, kp_t, mmat):
  return pl.pallas_call(
      _tc_body,
      grid_spec=pltpu.PrefetchScalarGridSpec(
          num_scalar_prefetch=1,
          grid=(_B,),
          in_specs=[
              pl.BlockSpec(memory_space=pl.ANY),
              pl.BlockSpec((1, 1, _PAD_SEL), lambda b, tref: (b, 0, 0)),
              pl.BlockSpec((_KP_IN, _KP_COLS), lambda b, tref: (0, 0)),
          ],
          out_specs=pl.BlockSpec(
              (1, _PAD_SEL, _KP_COLS), lambda b, tref: (b, 0, 0)),
          scratch_shapes=[
              pltpu.VMEM((_KP_IN, _N), jnp.float32),
              pltpu.VMEM((_KP_IN, _N), jnp.float32),
              pltpu.SemaphoreType.DMA,
              pltpu.SemaphoreType.DMA,
          ],
      ),
      out_shape=jax.ShapeDtypeStruct((_B, _PAD_SEL, _KP_COLS), jnp.float32),
  )(target_sizes, kp_t, nidx.reshape(_B, 1, _PAD_SEL), mmat)


def _build_mmat():
  m = np.zeros((_KP_IN, _KP_COLS), np.float32)
  for j in range(_KP_IN):
    m[j, 3 * (j // 2) + (j % 2)] = 1.0
  return m


@jax.jit
def _post_process(pred_logits, pred_keypoints, target_sizes):
  logits_t = pred_logits.reshape(_B, _NL)
  kp_t = jnp.transpose(pred_keypoints, (2, 0, 1))
  scores_p, labels_p, nidx = _sc_select(logits_t)
  mmat = jnp.asarray(_build_mmat())
  kp_p = _tc_gather(nidx, target_sizes, kp_t, mmat)
  return (scores_p[:, :NUM_SELECT],
          labels_p[:, :NUM_SELECT],
          kp_p[:, :NUM_SELECT, :])


def kernel(pred_logits, pred_keypoints, target_sizes):
  return _post_process(pred_logits, pred_keypoints, target_sizes)


# speedup vs baseline: 1.5836x; 1.2445x over previous
import functools

import jax
import jax.numpy as jnp
import numpy as np
from jax import lax
from jax.experimental import pallas as pl
from jax.experimental.pallas import tpu as pltpu
from jax.experimental.pallas import tpu_sc as plsc

NUM_SELECT = 60
NUM_BODY_POINTS = 17
_B = 32
_N = 20000
_C = 2
_NL = _N * _C
_NCH = _NL // 16
_CAP = 4096
_PAD_SEL = 64
_KP_IN = NUM_BODY_POINTS * 2
_KP_COLS = NUM_BODY_POINTS * 3
_GW = 160
_NEG = float("-inf")
_IMAX = 2**31 - 1


def _extract_top60(read_val, write_val, read_idx, nv):
  lane = lax.iota(jnp.int32, 16)

  def round_body(r, carry):
    s0, s1, s2, s3, i0, i1, i2, i3 = carry

    def max_body(k, acc):
      return jnp.maximum(acc, read_val(k))

    mx = lax.fori_loop(0, nv, max_body, jnp.full((16,), _NEG, jnp.float32))
    m = jnp.max(mx)

    def idx_body(k, acc):
      v = read_val(k)
      ii = read_idx(k)
      return jnp.minimum(acc, jnp.where(v == m, ii, _IMAX))

    mi_v = lax.fori_loop(0, nv, idx_body, jnp.full((16,), _IMAX, jnp.int32))
    mi = -jnp.max(-mi_v)

    def clear_body(k, c):
      v = read_val(k)
      ii = read_idx(k)
      write_val(k, jnp.where(ii == mi, _NEG, v))
      return c

    lax.fori_loop(0, nv, clear_body, 0)

    lane_hit = lane == (r & 15)
    slot = r >> 4
    mv = jnp.full((16,), m, jnp.float32)
    iv = jnp.full((16,), mi, jnp.int32)
    s0 = jnp.where(jnp.logical_and(lane_hit, slot == 0), mv, s0)
    s1 = jnp.where(jnp.logical_and(lane_hit, slot == 1), mv, s1)
    s2 = jnp.where(jnp.logical_and(lane_hit, slot == 2), mv, s2)
    s3 = jnp.where(jnp.logical_and(lane_hit, slot == 3), mv, s3)
    i0 = jnp.where(jnp.logical_and(lane_hit, slot == 0), iv, i0)
    i1 = jnp.where(jnp.logical_and(lane_hit, slot == 1), iv, i1)
    i2 = jnp.where(jnp.logical_and(lane_hit, slot == 2), iv, i2)
    i3 = jnp.where(jnp.logical_and(lane_hit, slot == 3), iv, i3)
    return s0, s1, s2, s3, i0, i1, i2, i3

  zf = jnp.zeros((16,), jnp.float32)
  zi = jnp.zeros((16,), jnp.int32)
  return lax.fori_loop(0, NUM_SELECT, round_body,
                       (zf, zf, zf, zf, zi, zi, zi, zi))


def _sc_body(logits_hbm, scores_hbm, labels_hbm, nidx_hbm,
             x_v, cv, ci, sc_v, lb_v, ni_v):
  b = lax.axis_index("s") * 2 + lax.axis_index("c")

  pltpu.sync_copy(logits_hbm.at[b], x_v)

  lane = lax.iota(jnp.int32, 16)
  negv = jnp.full((16,), _NEG, jnp.float32)

  def log_idx(k):
    p = lane + k * 16
    c = (p >= _N).astype(jnp.int32)
    return 2 * p - (_NL - 1) * c

  def p1(k, carry):
    r0, r1, r2, r3 = carry
    v = x_v[pl.ds(k * 16, 16)]
    b0 = jnp.maximum(r0, v)
    v1 = jnp.minimum(r0, v)
    b1 = jnp.maximum(r1, v1)
    v2 = jnp.minimum(r1, v1)
    b2 = jnp.maximum(r2, v2)
    v3 = jnp.minimum(r2, v2)
    b3 = jnp.maximum(r3, v3)
    return b0, b1, b2, b3

  _, _, _, r3 = lax.fori_loop(0, _NCH, p1, (negv, negv, negv, negv))
  t = -jnp.max(-r3)
  tv = jnp.full((16,), t, jnp.float32)

  def p2(k, pos):
    v = x_v[pl.ds(k * 16, 16)]
    m = v >= tv
    ps = jnp.minimum(pos, _CAP)
    plsc.store_compressed(cv.at[pl.ds(ps, 16)], v, mask=m)
    plsc.store_compressed(ci.at[pl.ds(ps, 16)], log_idx(k), mask=m)
    return pos + jnp.max(plsc.all_reduce_population_count(m))

  cnt = lax.fori_loop(0, _NCH, p2, 0)

  ps = jnp.minimum(cnt, _CAP)
  cv[pl.ds(ps, 16)] = negv
  ci[pl.ds(ps, 16)] = jnp.full((16,), _IMAX, jnp.int32)

  def main_path(_):
    nv = (jnp.minimum(cnt, _CAP) + 15) >> 4

    def rv(k):
      return cv[pl.ds(k * 16, 16)]

    def wv(k, x):
      cv[pl.ds(k * 16, 16)] = x

    def ri(k):
      return ci[pl.ds(k * 16, 16)]

    return _extract_top60(rv, wv, ri, nv)

  def slow_path(_):
    def rv(k):
      return x_v[pl.ds(k * 16, 16)]

    def wv(k, x):
      x_v[pl.ds(k * 16, 16)] = x

    return _extract_top60(rv, wv, log_idx, _NCH)

  s0, s1, s2, s3, i0, i1, i2, i3 = lax.cond(
      cnt <= _CAP, main_path, slow_path, 0)

  one = jnp.float32(1.0)
  for s, (svreg, ivreg) in enumerate(
      ((s0, i0), (s1, i1), (s2, i2), (s3, i3))):
    sc_v[pl.ds(s * 16, 16)] = one / (one + jnp.exp(-svreg))
    lb_v[pl.ds(s * 16, 16)] = ivreg & 1
    ni_v[pl.ds(s * 16, 16)] = ivreg >> 1

  pltpu.sync_copy(sc_v, scores_hbm.at[b])
  pltpu.sync_copy(lb_v, labels_hbm.at[b])
  pltpu.sync_copy(ni_v, nidx_hbm.at[b])


def _sc_select(logits_t):
  mesh = plsc.VectorSubcoreMesh(core_axis_name="c", subcore_axis_name="s")
  f = pl.kernel(
      _sc_body,
      out_type=(
          jax.ShapeDtypeStruct((_B, _PAD_SEL), jnp.float32),
          jax.ShapeDtypeStruct((_B, _PAD_SEL), jnp.int32),
          jax.ShapeDtypeStruct((_B, _PAD_SEL), jnp.int32),
      ),
      mesh=mesh,
      compiler_params=pltpu.CompilerParams(needs_layout_passes=False),
      scratch_types=[
          pltpu.VMEM((_NL,), jnp.float32),
          pltpu.VMEM((_CAP + 16,), jnp.float32),
          pltpu.VMEM((_CAP + 16,), jnp.int32),
          pltpu.VMEM((_PAD_SEL,), jnp.float32),
          pltpu.VMEM((_PAD_SEL,), jnp.int32),
          pltpu.VMEM((_PAD_SEL,), jnp.int32),
      ],
  )
  return f(logits_t)


def _tc_body(ts_smem, kp_hbm, nidx_ref, m_ref, out_ref,
             slab0, slab1, sem0, sem1):
  b = pl.program_id(0)

  @pl.when(b == 0)
  def _():
    pltpu.make_async_copy(kp_hbm.at[:, 0, :], slab0, sem0).start()

  def step(cur_slab, cur_sem, nxt_slab, nxt_sem):
    @pl.when(b + 1 < _B)
    def _():
      pltpu.make_async_copy(kp_hbm.at[:, b + 1, :], nxt_slab, nxt_sem).start()

    pltpu.make_async_copy(kp_hbm.at[:, b, :], cur_slab, cur_sem).wait()

    ks = lax.broadcasted_iota(jnp.int32, (_N, _PAD_SEL), 0)
    oh = (ks == nidx_ref[0]).astype(jnp.float32)
    cols = jax.lax.dot_general(
        cur_slab[...], oh, (((1,), (0,)), ((), ())),
        preferred_element_type=jnp.float32)

    rows = cols.T
    mm = jax.lax.dot_general(
        rows, m_ref[...], (((1,), (0,)), ((), ())),
        precision=lax.Precision.HIGHEST,
        preferred_element_type=jnp.float32)
    hh = ts_smem[b, 0]
    ww = ts_smem[b, 1]
    rem = lax.broadcasted_iota(jnp.int32, (_PAD_SEL, _KP_COLS), 1) % 3
    svec = jnp.where(rem == 0, ww, jnp.where(rem == 1, hh, 0.0))
    ones = jnp.where(rem == 2, 1.0, 0.0)
    out_ref[0] = mm * svec + ones

  @pl.when((b & 1) == 0)
  def _():
    step(slab0, sem0, slab1, sem1)

  @pl.when((b & 1) == 1)
  def _():
    step(slab1, sem1, slab0, sem0)


def _tc_gather(nidx, target_sizes, kp_t, mmat):
  return pl.pallas_call(
      _tc_body,
      grid_spec=pltpu.PrefetchScalarGridSpec(
          num_scalar_prefetch=1,
          grid=(_B,),
          in_specs=[
              pl.BlockSpec(memory_space=pl.ANY),
              pl.BlockSpec((1, 1, _PAD_SEL), lambda b, tref: (b, 0, 0)),
              pl.BlockSpec((_KP_IN, _KP_COLS), lambda b, tref: (0, 0)),
          ],
          out_specs=pl.BlockSpec(
              (1, _PAD_SEL, _KP_COLS), lambda b, tref: (b, 0, 0)),
          scratch_shapes=[
              pltpu.VMEM((_KP_IN, _N), jnp.float32),
              pltpu.VMEM((_KP_IN, _N), jnp.float32),
              pltpu.SemaphoreType.DMA,
              pltpu.SemaphoreType.DMA,
          ],
      ),
      out_shape=jax.ShapeDtypeStruct((_B, _PAD_SEL, _KP_COLS), jnp.float32),
  )(target_sizes, kp_t, nidx.reshape(_B, 1, _PAD_SEL), mmat)


def _build_mmat():
  m = np.zeros((_KP_IN, _KP_COLS), np.float32)
  for j in range(_KP_IN):
    m[j, 3 * (j // 2) + (j % 2)] = 1.0
  return m


@jax.jit
def _post_process(pred_logits, pred_keypoints, target_sizes):
  logits_t = jnp.transpose(pred_logits, (0, 2, 1)).reshape(_B, _NL)
  kp_t = jnp.transpose(pred_keypoints, (2, 0, 1))
  scores_p, labels_p, nidx = _sc_select(logits_t)
  mmat = jnp.asarray(_build_mmat())
  kp_p = _tc_gather(nidx, target_sizes, kp_t, mmat)
  return (scores_p[:, :NUM_SELECT],
          labels_p[:, :NUM_SELECT],
          kp_p[:, :NUM_SELECT, :])


def kernel(pred_logits, pred_keypoints, target_sizes):
  return _post_process(pred_logits, pred_keypoints, target_sizes)


# speedup vs baseline: 1.8241x; 1.1519x over previous
import functools

import jax
import jax.numpy as jnp
import numpy as np
from jax import lax
from jax.experimental import pallas as pl
from jax.experimental.pallas import tpu as pltpu
from jax.experimental.pallas import tpu_sc as plsc

NUM_SELECT = 60
NUM_BODY_POINTS = 17
_B = 32
_N = 20000
_C = 2
_NL = _N * _C
_NCH = _NL // 16
_CAP = 4096
_PAD_SEL = 64
_KP_IN = NUM_BODY_POINTS * 2
_KP_COLS = NUM_BODY_POINTS * 3
_GW = 160
_NEG = float("-inf")
_IMAX = 2**31 - 1


def _extract_top60(read_val, write_val, read_idx, nv):
  lane = lax.iota(jnp.int32, 16)

  def round_body(r, carry):
    s0, s1, s2, s3, i0, i1, i2, i3 = carry

    def max_body(k, acc):
      return jnp.maximum(acc, read_val(k))

    mx = lax.fori_loop(0, nv, max_body, jnp.full((16,), _NEG, jnp.float32))
    m = jnp.max(mx)

    def idx_body(k, acc):
      v = read_val(k)
      ii = read_idx(k)
      return jnp.minimum(acc, jnp.where(v == m, ii, _IMAX))

    mi_v = lax.fori_loop(0, nv, idx_body, jnp.full((16,), _IMAX, jnp.int32))
    mi = -jnp.max(-mi_v)

    def clear_body(k, c):
      v = read_val(k)
      ii = read_idx(k)
      write_val(k, jnp.where(ii == mi, _NEG, v))
      return c

    lax.fori_loop(0, nv, clear_body, 0)

    lane_hit = lane == (r & 15)
    slot = r >> 4
    mv = jnp.full((16,), m, jnp.float32)
    iv = jnp.full((16,), mi, jnp.int32)
    s0 = jnp.where(jnp.logical_and(lane_hit, slot == 0), mv, s0)
    s1 = jnp.where(jnp.logical_and(lane_hit, slot == 1), mv, s1)
    s2 = jnp.where(jnp.logical_and(lane_hit, slot == 2), mv, s2)
    s3 = jnp.where(jnp.logical_and(lane_hit, slot == 3), mv, s3)
    i0 = jnp.where(jnp.logical_and(lane_hit, slot == 0), iv, i0)
    i1 = jnp.where(jnp.logical_and(lane_hit, slot == 1), iv, i1)
    i2 = jnp.where(jnp.logical_and(lane_hit, slot == 2), iv, i2)
    i3 = jnp.where(jnp.logical_and(lane_hit, slot == 3), iv, i3)
    return s0, s1, s2, s3, i0, i1, i2, i3

  zf = jnp.zeros((16,), jnp.float32)
  zi = jnp.zeros((16,), jnp.int32)
  return lax.fori_loop(0, NUM_SELECT, round_body,
                       (zf, zf, zf, zf, zi, zi, zi, zi))


def _sc_body(logits_hbm, scores_hbm, labels_hbm, nidx_hbm,
             x_v, cv, ci, sc_v, lb_v, ni_v):
  b = lax.axis_index("s") * 2 + lax.axis_index("c")

  pltpu.sync_copy(logits_hbm.at[b], x_v)

  lane = lax.iota(jnp.int32, 16)
  negv = jnp.full((16,), _NEG, jnp.float32)

  def log_idx(k):
    p = lane + k * 16
    c = (p >= _N).astype(jnp.int32)
    return 2 * p - (_NL - 1) * c

  def p1(q, carry):
    r0, r1, r2, r3 = carry
    for u in range(4):
      v = x_v[pl.ds((q * 4 + u) * 16, 16)]
      b0 = jnp.maximum(r0, v)
      v1 = jnp.minimum(r0, v)
      b1 = jnp.maximum(r1, v1)
      v2 = jnp.minimum(r1, v1)
      b2 = jnp.maximum(r2, v2)
      v3 = jnp.minimum(r2, v2)
      r0, r1, r2, r3 = b0, b1, b2, jnp.maximum(r3, v3)
    return r0, r1, r2, r3

  _, _, _, r3 = lax.fori_loop(0, _NCH // 4, p1, (negv, negv, negv, negv))
  t = -jnp.max(-r3)
  tv = jnp.full((16,), t, jnp.float32)

  def p2(q, pos):
    ms = []
    for u in range(4):
      k = q * 4 + u
      v = x_v[pl.ds(k * 16, 16)]
      m = v >= tv
      ms.append((k, v, m, plsc.all_reduce_population_count(m)))
    for k, v, m, pc in ms:
      ps = jnp.minimum(pos, _CAP)
      plsc.store_compressed(cv.at[pl.ds(ps, 16)], v, mask=m)
      plsc.store_compressed(ci.at[pl.ds(ps, 16)], log_idx(k), mask=m)
      pos = pos + jnp.max(pc)
    return pos

  cnt = lax.fori_loop(0, _NCH // 4, p2, 0)

  ps = jnp.minimum(cnt, _CAP)
  cv[pl.ds(ps, 16)] = negv
  ci[pl.ds(ps, 16)] = jnp.full((16,), _IMAX, jnp.int32)

  def main_path(_):
    nv = (jnp.minimum(cnt, _CAP) + 15) >> 4

    def rv(k):
      return cv[pl.ds(k * 16, 16)]

    def wv(k, x):
      cv[pl.ds(k * 16, 16)] = x

    def ri(k):
      return ci[pl.ds(k * 16, 16)]

    return _extract_top60(rv, wv, ri, nv)

  def slow_path(_):
    def rv(k):
      return x_v[pl.ds(k * 16, 16)]

    def wv(k, x):
      x_v[pl.ds(k * 16, 16)] = x

    return _extract_top60(rv, wv, log_idx, _NCH)

  s0, s1, s2, s3, i0, i1, i2, i3 = lax.cond(
      cnt <= _CAP, main_path, slow_path, 0)

  one = jnp.float32(1.0)
  for s, (svreg, ivreg) in enumerate(
      ((s0, i0), (s1, i1), (s2, i2), (s3, i3))):
    sc_v[pl.ds(s * 16, 16)] = one / (one + jnp.exp(-svreg))
    lb_v[pl.ds(s * 16, 16)] = ivreg & 1
    ni_v[pl.ds(s * 16, 16)] = ivreg >> 1

  pltpu.sync_copy(sc_v, scores_hbm.at[b])
  pltpu.sync_copy(lb_v, labels_hbm.at[b])
  pltpu.sync_copy(ni_v, nidx_hbm.at[b])


def _sc_select(logits_t):
  mesh = plsc.VectorSubcoreMesh(core_axis_name="c", subcore_axis_name="s")
  f = pl.kernel(
      _sc_body,
      out_type=(
          jax.ShapeDtypeStruct((_B, _PAD_SEL), jnp.float32),
          jax.ShapeDtypeStruct((_B, _PAD_SEL), jnp.int32),
          jax.ShapeDtypeStruct((_B, _PAD_SEL), jnp.int32),
      ),
      mesh=mesh,
      compiler_params=pltpu.CompilerParams(needs_layout_passes=False),
      scratch_types=[
          pltpu.VMEM((_NL,), jnp.float32),
          pltpu.VMEM((_CAP + 16,), jnp.float32),
          pltpu.VMEM((_CAP + 16,), jnp.int32),
          pltpu.VMEM((_PAD_SEL,), jnp.float32),
          pltpu.VMEM((_PAD_SEL,), jnp.int32),
          pltpu.VMEM((_PAD_SEL,), jnp.int32),
      ],
  )
  return f(logits_t)


def _tc_body(ts_smem, kp_hbm, nidx_ref, m_ref, out_ref,
             slab0, slab1, sem0, sem1):
  b = pl.program_id(0)

  @pl.when(b == 0)
  def _():
    pltpu.make_async_copy(kp_hbm.at[:, 0, :], slab0, sem0).start()

  def step(cur_slab, cur_sem, nxt_slab, nxt_sem):
    @pl.when(b + 1 < _B)
    def _():
      pltpu.make_async_copy(kp_hbm.at[:, b + 1, :], nxt_slab, nxt_sem).start()

    pltpu.make_async_copy(kp_hbm.at[:, b, :], cur_slab, cur_sem).wait()

    ks = lax.broadcasted_iota(jnp.int32, (_N, _PAD_SEL), 0)
    oh = (ks == nidx_ref[0]).astype(jnp.float32)
    cols = jax.lax.dot_general(
        cur_slab[...], oh, (((1,), (0,)), ((), ())),
        preferred_element_type=jnp.float32)

    rows = cols.T
    mm = jax.lax.dot_general(
        rows, m_ref[...], (((1,), (0,)), ((), ())),
        precision=lax.Precision.HIGHEST,
        preferred_element_type=jnp.float32)
    hh = ts_smem[b, 0]
    ww = ts_smem[b, 1]
    rem = lax.broadcasted_iota(jnp.int32, (_PAD_SEL, _KP_COLS), 1) % 3
    svec = jnp.where(rem == 0, ww, jnp.where(rem == 1, hh, 0.0))
    ones = jnp.where(rem == 2, 1.0, 0.0)
    out_ref[0] = mm * svec + ones

  @pl.when((b & 1) == 0)
  def _():
    step(slab0, sem0, slab1, sem1)

  @pl.when((b & 1) == 1)
  def _():
    step(slab1, sem1, slab0, sem0)


def _tc_gather(nidx, target_sizes, kp_t, mmat):
  return pl.pallas_call(
      _tc_body,
      grid_spec=pltpu.PrefetchScalarGridSpec(
          num_scalar_prefetch=1,
          grid=(_B,),
          in_specs=[
              pl.BlockSpec(memory_space=pl.ANY),
              pl.BlockSpec((1, 1, _PAD_SEL), lambda b, tref: (b, 0, 0)),
              pl.BlockSpec((_KP_IN, _KP_COLS), lambda b, tref: (0, 0)),
          ],
          out_specs=pl.BlockSpec(
              (1, _PAD_SEL, _KP_COLS), lambda b, tref: (b, 0, 0)),
          scratch_shapes=[
              pltpu.VMEM((_KP_IN, _N), jnp.float32),
              pltpu.VMEM((_KP_IN, _N), jnp.float32),
              pltpu.SemaphoreType.DMA,
              pltpu.SemaphoreType.DMA,
          ],
      ),
      out_shape=jax.ShapeDtypeStruct((_B, _PAD_SEL, _KP_COLS), jnp.float32),
  )(target_sizes, kp_t, nidx.reshape(_B, 1, _PAD_SEL), mmat)


def _build_mmat():
  m = np.zeros((_KP_IN, _KP_COLS), np.float32)
  for j in range(_KP_IN):
    m[j, 3 * (j // 2) + (j % 2)] = 1.0
  return m


@jax.jit
def _post_process(pred_logits, pred_keypoints, target_sizes):
  logits_t = jnp.transpose(pred_logits, (0, 2, 1)).reshape(_B, _NL)
  kp_t = jnp.transpose(pred_keypoints, (2, 0, 1))
  scores_p, labels_p, nidx = _sc_select(logits_t)
  mmat = jnp.asarray(_build_mmat())
  kp_p = _tc_gather(nidx, target_sizes, kp_t, mmat)
  return (scores_p[:, :NUM_SELECT],
          labels_p[:, :NUM_SELECT],
          kp_p[:, :NUM_SELECT, :])


def kernel(pred_logits, pred_keypoints, target_sizes):
  return _post_process(pred_logits, pred_keypoints, target_sizes)
